# baseline (speedup 1.0000x reference)
import functools

import jax
import jax.numpy as jnp
from jax import lax
from jax.experimental import pallas as pl
from jax.experimental.pallas import tpu as pltpu
from jax.experimental.pallas import tpu_sc as plsc

N = 50000
E = 800000
IN = 128
H = 64
NG = 64
G = 107

RB = 5000
NBLK = N // RB

_f32 = jnp.float32
_NEG = -1e30


def _oh(batch_blk):
    segs = lax.broadcasted_iota(jnp.int32, (1, NG), 1)
    return (batch_blk == segs).astype(_f32)


def _k1_body(x_ref, win_ref, bin_ref, w0_ref, deg_ref,
             h_ref, xw_ref, dis_ref, sn_ref):
    h = jnp.maximum(jnp.dot(x_ref[...], win_ref[...],
                            preferred_element_type=_f32, precision=lax.Precision.HIGHEST) + bin_ref[...], 0.0)
    h_ref[...] = h
    xw = jnp.dot(h, w0_ref[...], preferred_element_type=_f32, precision=lax.Precision.HIGHEST)
    xw_ref[0] = xw[:, :32]
    xw_ref[1] = xw[:, 32:]
    deg = deg_ref[...]
    dis = jnp.where(deg > 0, lax.rsqrt(jnp.where(deg > 0, deg, 1.0)), 0.0)
    dis_ref[...] = dis
    sn_ref[...] = 2.0 * dis * dis


def _k1(x, W_in, b_in, W0, deg):
    return pl.pallas_call(
        _k1_body,
        grid=(NBLK,),
        in_specs=[
            pl.BlockSpec((RB, IN), lambda i: (i, 0)),
            pl.BlockSpec((IN, H), lambda i: (0, 0)),
            pl.BlockSpec((1, H), lambda i: (0, 0)),
            pl.BlockSpec((H, H), lambda i: (0, 0)),
            pl.BlockSpec((RB, 1), lambda i: (i, 0)),
        ],
        out_specs=[
            pl.BlockSpec((RB, H), lambda i: (i, 0)),
            pl.BlockSpec((2, RB, 32), lambda i: (0, i, 0)),
            pl.BlockSpec((RB, 1), lambda i: (i, 0)),
            pl.BlockSpec((RB, 1), lambda i: (i, 0)),
        ],
        out_shape=[
            jax.ShapeDtypeStruct((N, H), _f32),
            jax.ShapeDtypeStruct((2, N, 32), _f32),
            jax.ShapeDtypeStruct((N, 1), _f32),
            jax.ShapeDtypeStruct((N, 1), _f32),
        ],
    )(x, W_in, b_in, W0, deg)


def _k2_body(scat_ref, xw_ref, sn_ref, b_ref, out_ref, sums_ref, acc_ref):
    i = pl.program_id(0)
    scat = jnp.concatenate([scat_ref[0], scat_ref[1]], axis=1)
    xw = jnp.concatenate([xw_ref[0], xw_ref[1]], axis=1)
    out = scat + sn_ref[...] * xw + b_ref[...]
    out_ref[...] = out
    ps = jnp.sum(out, axis=0)[None]
    ps2 = jnp.sum(out * out, axis=0)[None]
    part = jnp.concatenate([ps, ps2, jnp.zeros((6, H), _f32)], axis=0)

    @pl.when(i == 0)
    def _():
        acc_ref[...] = jnp.zeros((8, H), _f32)

    acc_ref[...] += part
    @pl.when(i == NBLK - 1)
    def _():
        sums_ref[...] = acc_ref[...]


def _k2(scat, xw, sn, b_conv_i):
    return pl.pallas_call(
        _k2_body,
        grid=(NBLK,),
        in_specs=[
            pl.BlockSpec((2, RB, 32), lambda i: (0, i, 0)),
            pl.BlockSpec((2, RB, 32), lambda i: (0, i, 0)),
            pl.BlockSpec((RB, 1), lambda i: (i, 0)),
            pl.BlockSpec((1, H), lambda i: (0, 0)),
        ],
        out_specs=[
            pl.BlockSpec((RB, H), lambda i: (i, 0)),
            pl.BlockSpec((8, H), lambda i: (0, 0)),
        ],
        out_shape=[
            jax.ShapeDtypeStruct((N, H), _f32),
            jax.ShapeDtypeStruct((8, H), _f32),
        ],
        scratch_shapes=[pltpu.VMEM((8, H), _f32)],
    )(scat, xw, sn, b_conv_i)


def _k23_body(scat_ref, sn_ref, b_ref, prev_ref, g_ref, bb_ref,
              wi_ref, wn_ref, h_ref, xwn_ref, acc_ref):
    i = pl.program_id(0)
    prev = prev_ref[...]
    xw = jnp.dot(prev, wi_ref[...], preferred_element_type=_f32,
                 precision=lax.Precision.HIGHEST)
    out = scat_ref[...] + sn_ref[...] * xw + b_ref[...]

    @pl.when(i < NBLK)
    def _():
        ps = jnp.sum(out, axis=0)[None]
        ps2 = jnp.sum(out * out, axis=0)[None]
        part = jnp.concatenate([ps, ps2, jnp.zeros((6, H), _f32)], axis=0)

        @pl.when(i == 0)
        def _():
            acc_ref[...] = jnp.zeros((8, H), _f32)

        acc_ref[...] += part

    @pl.when(i >= NBLK)
    def _():
        sums = acc_ref[...]
        m = sums[0:1] * (1.0 / N)
        v = sums[1:2] * (1.0 / N) - m * m
        inv = lax.rsqrt(v + 1e-5)
        hn = (out - m) * inv * g_ref[...] + bb_ref[...] + prev
        h_ref[...] = hn
        xwn = jnp.dot(hn, wn_ref[...], preferred_element_type=_f32,
                      precision=lax.Precision.HIGHEST)
        xwn_ref[0] = xwn[:, :32]
        xwn_ref[1] = xwn[:, 32:]


def _k23(scat, sn, b_conv_i, prev, bn_g, bn_b, W_i, W_next):
    pj = lambda i: (lax.rem(i, NBLK), 0)
    p2 = lambda i: (jnp.maximum(i - NBLK, 0), 0)
    c0 = lambda i: (0, 0)
    return pl.pallas_call(
        _k23_body,
        grid=(2 * NBLK,),
        in_specs=[
            pl.BlockSpec((RB, H), pj),
            pl.BlockSpec((RB, 1), pj),
            pl.BlockSpec((1, H), c0),
            pl.BlockSpec((RB, H), pj),
            pl.BlockSpec((1, H), c0),
            pl.BlockSpec((1, H), c0),
            pl.BlockSpec((H, H), c0),
            pl.BlockSpec((H, H), c0),
        ],
        out_specs=[
            pl.BlockSpec((RB, H), p2),
            pl.BlockSpec((2, RB, 32), lambda i: (0, jnp.maximum(i - NBLK, 0),
                                                 0)),
        ],
        out_shape=[
            jax.ShapeDtypeStruct((N, H), _f32),
            jax.ShapeDtypeStruct((2, N, 32), _f32),
        ],
        scratch_shapes=[pltpu.VMEM((8, H), _f32)],
    )(scat, sn, b_conv_i, prev, bn_g, bn_b, W_i, W_next)


def _k3_body(out_ref, prev_ref, sums_ref, g_ref, bb_ref, wn_ref,
             h_ref, xw_ref):
    sums = sums_ref[...]
    m = sums[0:1] * (1.0 / N)
    v = sums[1:2] * (1.0 / N) - m * m
    inv = lax.rsqrt(v + 1e-5)
    hn = (out_ref[...] - m) * inv * g_ref[...] + bb_ref[...] + prev_ref[...]
    h_ref[...] = hn
    xw = jnp.dot(hn, wn_ref[...], preferred_element_type=_f32, precision=lax.Precision.HIGHEST)
    xw_ref[0] = xw[:, :32]
    xw_ref[1] = xw[:, 32:]


def _k3(out, prev, sums, bn_g, bn_b, W_next):
    return pl.pallas_call(
        _k3_body,
        grid=(NBLK,),
        in_specs=[
            pl.BlockSpec((RB, H), lambda i: (i, 0)),
            pl.BlockSpec((RB, H), lambda i: (i, 0)),
            pl.BlockSpec((8, H), lambda i: (0, 0)),
            pl.BlockSpec((1, H), lambda i: (0, 0)),
            pl.BlockSpec((1, H), lambda i: (0, 0)),
            pl.BlockSpec((H, H), lambda i: (0, 0)),
        ],
        out_specs=[
            pl.BlockSpec((RB, H), lambda i: (i, 0)),
            pl.BlockSpec((2, RB, 32), lambda i: (0, i, 0)),
        ],
        out_shape=[
            jax.ShapeDtypeStruct((N, H), _f32),
            jax.ShapeDtypeStruct((2, N, 32), _f32),
        ],
    )(out, prev, sums, bn_g, bn_b, W_next)


def _k23l_body(scat_ref, sn_ref, b_ref, prev_ref, g_ref, bb_ref, wi_ref,
               gi_ref, batch_ref, wgh_ref, wgg_ref, bgin_ref, wg0_ref,
               bg0_ref, wg1_ref, bg1_ref, wgo_ref, bgo_ref,
               h_ref, score_ref, m_ref, acc_ref, accm_ref):
    i = pl.program_id(0)
    prev = prev_ref[...]
    xw = jnp.dot(prev, wi_ref[...], preferred_element_type=_f32,
                 precision=lax.Precision.HIGHEST)
    out = scat_ref[...] + sn_ref[...] * xw + b_ref[...]

    @pl.when(i < NBLK)
    def _():
        ps = jnp.sum(out, axis=0)[None]
        ps2 = jnp.sum(out * out, axis=0)[None]
        part = jnp.concatenate([ps, ps2, jnp.zeros((6, H), _f32)], axis=0)

        @pl.when(i == 0)
        def _():
            acc_ref[...] = jnp.zeros((8, H), _f32)

        acc_ref[...] += part

    @pl.when(i >= NBLK)
    def _():
        j = i - NBLK
        sums = acc_ref[...]
        m = sums[0:1] * (1.0 / N)
        v = sums[1:2] * (1.0 / N) - m * m
        inv = lax.rsqrt(v + 1e-5)
        hn = (out - m) * inv * g_ref[...] + bb_ref[...] + prev
        h_ref[...] = hn
        g1 = jnp.maximum(
            jnp.dot(hn, wgh_ref[...], preferred_element_type=_f32,
                    precision=lax.Precision.HIGHEST)
            + jnp.dot(gi_ref[...], wgg_ref[...], preferred_element_type=_f32,
                      precision=lax.Precision.HIGHEST)
            + bgin_ref[...], 0.0)
        g2 = jnp.maximum(
            jnp.dot(g1, wg0_ref[...], preferred_element_type=_f32,
                    precision=lax.Precision.HIGHEST) + bg0_ref[...], 0.0)
        g3 = jnp.maximum(
            jnp.dot(g2, wg1_ref[...], preferred_element_type=_f32,
                    precision=lax.Precision.HIGHEST) + bg1_ref[...], 0.0)
        score = jnp.dot(g3, wgo_ref[...], preferred_element_type=_f32,
                        precision=lax.Precision.HIGHEST) + bgo_ref[...]
        score_ref[...] = score
        oh = _oh(batch_ref[...])
        masked = jnp.where(oh > 0, score, _NEG)
        mpart = jnp.max(masked, axis=0)[None]

        @pl.when(j == 0)
        def _():
            accm_ref[...] = jnp.full((8, NG), _NEG, _f32)

        accm_ref[0:1] = jnp.maximum(accm_ref[0:1], mpart)
        @pl.when(j == NBLK - 1)
        def _():
            m_ref[...] = accm_ref[0:1]


def _k23l(scat, sn, b_conv_i, prev, bn_g, bn_b, W_i, gi, batch,
          Wg_h, Wg_g, b_gin, Wg0, bg0, Wg1, bg1, Wgo, bgo):
    pj = lambda i: (lax.rem(i, NBLK), 0)
    p2 = lambda i: (jnp.maximum(i - NBLK, 0), 0)
    c0 = lambda i: (0, 0)
    return pl.pallas_call(
        _k23l_body,
        grid=(2 * NBLK,),
        in_specs=[
            pl.BlockSpec((RB, H), pj),
            pl.BlockSpec((RB, 1), pj),
            pl.BlockSpec((1, H), c0),
            pl.BlockSpec((RB, H), pj),
            pl.BlockSpec((1, H), c0),
            pl.BlockSpec((1, H), c0),
            pl.BlockSpec((H, H), c0),
            pl.BlockSpec((RB, G), p2),
            pl.BlockSpec((RB, 1), p2),
            pl.BlockSpec((H, H), c0),
            pl.BlockSpec((G, H), c0),
            pl.BlockSpec((1, H), c0),
            pl.BlockSpec((H, H), c0),
            pl.BlockSpec((1, H), c0),
            pl.BlockSpec((H, H), c0),
            pl.BlockSpec((1, H), c0),
            pl.BlockSpec((H, 1), c0),
            pl.BlockSpec((1, 1), c0),
        ],
        out_specs=[
            pl.BlockSpec((RB, H), p2),
            pl.BlockSpec((RB, 1), p2),
            pl.BlockSpec((1, NG), c0),
        ],
        out_shape=[
            jax.ShapeDtypeStruct((N, H), _f32),
            jax.ShapeDtypeStruct((N, 1), _f32),
            jax.ShapeDtypeStruct((1, NG), _f32),
        ],
        scratch_shapes=[pltpu.VMEM((8, H), _f32), pltpu.VMEM((8, NG), _f32)],
    )(scat, sn, b_conv_i, prev, bn_g, bn_b, W_i, gi, batch,
      Wg_h, Wg_g, b_gin, Wg0, bg0, Wg1, bg1, Wgo, bgo)


def _k3l_body(out_ref, prev_ref, sums_ref, g_ref, bb_ref, gi_ref, batch_ref,
              wgh_ref, wgg_ref, bgin_ref, wg0_ref, bg0_ref, wg1_ref, bg1_ref,
              wgo_ref, bgo_ref,
              h_ref, score_ref, m_ref, acc_ref):
    i = pl.program_id(0)
    sums = sums_ref[...]
    m = sums[0:1] * (1.0 / N)
    v = sums[1:2] * (1.0 / N) - m * m
    inv = lax.rsqrt(v + 1e-5)
    hn = (out_ref[...] - m) * inv * g_ref[...] + bb_ref[...] + prev_ref[...]
    h_ref[...] = hn
    g1 = jnp.maximum(
        jnp.dot(hn, wgh_ref[...], preferred_element_type=_f32, precision=lax.Precision.HIGHEST)
        + jnp.dot(gi_ref[...], wgg_ref[...], preferred_element_type=_f32, precision=lax.Precision.HIGHEST)
        + bgin_ref[...], 0.0)
    g2 = jnp.maximum(
        jnp.dot(g1, wg0_ref[...], preferred_element_type=_f32, precision=lax.Precision.HIGHEST) + bg0_ref[...],
        0.0)
    g3 = jnp.maximum(
        jnp.dot(g2, wg1_ref[...], preferred_element_type=_f32, precision=lax.Precision.HIGHEST) + bg1_ref[...],
        0.0)
    score = jnp.dot(g3, wgo_ref[...], preferred_element_type=_f32, precision=lax.Precision.HIGHEST) \
        + bgo_ref[...]
    score_ref[...] = score
    oh = _oh(batch_ref[...])
    masked = jnp.where(oh > 0, score, _NEG)
    mpart = jnp.max(masked, axis=0)[None]

    @pl.when(i == 0)
    def _():
        acc_ref[...] = jnp.full((8, NG), _NEG, _f32)

    acc_ref[0:1] = jnp.maximum(acc_ref[0:1], mpart)
    @pl.when(i == NBLK - 1)
    def _():
        m_ref[...] = acc_ref[0:1]


def _k3l(out, prev, sums, bn_g, bn_b, gi, batch, Wg_h, Wg_g, b_gin,
         Wg0, bg0, Wg1, bg1, Wgo, bgo):
    return pl.pallas_call(
        _k3l_body,
        grid=(NBLK,),
        in_specs=[
            pl.BlockSpec((RB, H), lambda i: (i, 0)),
            pl.BlockSpec((RB, H), lambda i: (i, 0)),
            pl.BlockSpec((8, H), lambda i: (0, 0)),
            pl.BlockSpec((1, H), lambda i: (0, 0)),
            pl.BlockSpec((1, H), lambda i: (0, 0)),
            pl.BlockSpec((RB, G), lambda i: (i, 0)),
            pl.BlockSpec((RB, 1), lambda i: (i, 0)),
            pl.BlockSpec((H, H), lambda i: (0, 0)),
            pl.BlockSpec((G, H), lambda i: (0, 0)),
            pl.BlockSpec((1, H), lambda i: (0, 0)),
            pl.BlockSpec((H, H), lambda i: (0, 0)),
            pl.BlockSpec((1, H), lambda i: (0, 0)),
            pl.BlockSpec((H, H), lambda i: (0, 0)),
            pl.BlockSpec((1, H), lambda i: (0, 0)),
            pl.BlockSpec((H, 1), lambda i: (0, 0)),
            pl.BlockSpec((1, 1), lambda i: (0, 0)),
        ],
        out_specs=[
            pl.BlockSpec((RB, H), lambda i: (i, 0)),
            pl.BlockSpec((RB, 1), lambda i: (i, 0)),
            pl.BlockSpec((1, NG), lambda i: (0, 0)),
        ],
        out_shape=[
            jax.ShapeDtypeStruct((N, H), _f32),
            jax.ShapeDtypeStruct((N, 1), _f32),
            jax.ShapeDtypeStruct((1, NG), _f32),
        ],
        scratch_shapes=[pltpu.VMEM((8, NG), _f32)],
    )(out, prev, sums, bn_g, bn_b, gi, batch, Wg_h, Wg_g, b_gin,
      Wg0, bg0, Wg1, bg1, Wgo, bgo)


def _k4_body(score_ref, batch_ref, m_ref, h_ref, bih_ref, bhh_ref,
             ex_ref, s_ref, d1_ref, acc_ref):
    i = pl.program_id(0)
    oh = _oh(batch_ref[...])
    m = m_ref[...]
    mc = jnp.where(m < -1e29, 0.0, m)
    mb = jnp.sum(oh * mc, axis=1, keepdims=True)
    ex = jnp.exp(score_ref[...] - mb)
    ex_ref[...] = ex
    spart = jnp.sum(oh * ex, axis=0)[None]

    gates = bih_ref[...] + bhh_ref[...]
    ii, ff, gg, oo = jnp.split(gates, 4, axis=1)
    cc1 = jax.nn.sigmoid(ii) * jnp.tanh(gg)
    hh1 = jax.nn.sigmoid(oo) * jnp.tanh(cc1)
    d1_ref[...] = jnp.sum(h_ref[...] * hh1, axis=1, keepdims=True)

    @pl.when(i == 0)
    def _():
        acc_ref[...] = jnp.zeros((8, NG), _f32)

    acc_ref[0:1] += spart
    @pl.when(i == NBLK - 1)
    def _():
        s_ref[...] = acc_ref[0:1]


def _k4(score, batch, m, h5, b_ih, b_hh):
    return pl.pallas_call(
        _k4_body,
        grid=(NBLK,),
        in_specs=[
            pl.BlockSpec((RB, 1), lambda i: (i, 0)),
            pl.BlockSpec((RB, 1), lambda i: (i, 0)),
            pl.BlockSpec((1, NG), lambda i: (0, 0)),
            pl.BlockSpec((RB, H), lambda i: (i, 0)),
            pl.BlockSpec((1, 4 * H), lambda i: (0, 0)),
            pl.BlockSpec((1, 4 * H), lambda i: (0, 0)),
        ],
        out_specs=[
            pl.BlockSpec((RB, 1), lambda i: (i, 0)),
            pl.BlockSpec((1, NG), lambda i: (0, 0)),
            pl.BlockSpec((RB, 1), lambda i: (i, 0)),
        ],
        out_shape=[
            jax.ShapeDtypeStruct((N, 1), _f32),
            jax.ShapeDtypeStruct((1, NG), _f32),
            jax.ShapeDtypeStruct((N, 1), _f32),
        ],
        scratch_shapes=[pltpu.VMEM((8, NG), _f32)],
    )(score, batch, m, h5, b_ih, b_hh)


def _k5_body(ex_ref, s_ref, batch_ref, h_ref, d1_ref,
             hw_ref, e1_ref, m1_ref, acc_ref):
    i = pl.program_id(0)
    oh = _oh(batch_ref[...])
    sb = jnp.sum(oh * s_ref[...], axis=1, keepdims=True)
    nw = ex_ref[...] / (sb + 1e-16)
    hw = h_ref[...] * nw
    hw_ref[...] = hw
    e1 = nw * d1_ref[...]
    e1_ref[...] = e1
    masked = jnp.where(oh > 0, e1, _NEG)
    mpart = jnp.max(masked, axis=0)[None]

    @pl.when(i == 0)
    def _():
        acc_ref[...] = jnp.full((8, NG), _NEG, _f32)

    acc_ref[0:1] = jnp.maximum(acc_ref[0:1], mpart)
    @pl.when(i == NBLK - 1)
    def _():
        m1_ref[...] = acc_ref[0:1]


def _k5(ex, s, batch, h5, d1):
    return pl.pallas_call(
        _k5_body,
        grid=(NBLK,),
        in_specs=[
            pl.BlockSpec((RB, 1), lambda i: (i, 0)),
            pl.BlockSpec((1, NG), lambda i: (0, 0)),
            pl.BlockSpec((RB, 1), lambda i: (i, 0)),
            pl.BlockSpec((RB, H), lambda i: (i, 0)),
            pl.BlockSpec((RB, 1), lambda i: (i, 0)),
        ],
        out_specs=[
            pl.BlockSpec((RB, H), lambda i: (i, 0)),
            pl.BlockSpec((RB, 1), lambda i: (i, 0)),
            pl.BlockSpec((1, NG), lambda i: (0, 0)),
        ],
        out_shape=[
            jax.ShapeDtypeStruct((N, H), _f32),
            jax.ShapeDtypeStruct((N, 1), _f32),
            jax.ShapeDtypeStruct((1, NG), _f32),
        ],
        scratch_shapes=[pltpu.VMEM((8, NG), _f32)],
    )(ex, s, batch, h5, d1)


def _k6_body(e_ref, m_ref, batch_ref, hw_ref, s_ref, ru_ref,
             accs_ref, accr_ref):
    i = pl.program_id(0)
    oh = _oh(batch_ref[...])
    m = m_ref[...]
    mc = jnp.where(m < -1e29, 0.0, m)
    mb = jnp.sum(oh * mc, axis=1, keepdims=True)
    ex = jnp.exp(e_ref[...] - mb)
    spart = jnp.sum(oh * ex, axis=0)[None]
    rupart = lax.dot_general(oh, ex * hw_ref[...],
                             (((0,), (0,)), ((), ())),
                             preferred_element_type=_f32, precision=lax.Precision.HIGHEST)

    @pl.when(i == 0)
    def _():
        accs_ref[...] = jnp.zeros((8, NG), _f32)
        accr_ref[...] = jnp.zeros((NG, H), _f32)

    accs_ref[0:1] += spart
    accr_ref[...] += rupart
    @pl.when(i == NBLK - 1)
    def _():
        s_ref[...] = accs_ref[0:1]
        ru_ref[...] = accr_ref[...]


def _k6(e, m, batch, hw):
    return pl.pallas_call(
        _k6_body,
        grid=(NBLK,),
        in_specs=[
            pl.BlockSpec((RB, 1), lambda i: (i, 0)),
            pl.BlockSpec((1, NG), lambda i: (0, 0)),
            pl.BlockSpec((RB, 1), lambda i: (i, 0)),
            pl.BlockSpec((RB, H), lambda i: (i, 0)),
        ],
        out_specs=[
            pl.BlockSpec((1, NG), lambda i: (0, 0)),
            pl.BlockSpec((NG, H), lambda i: (0, 0)),
        ],
        out_shape=[
            jax.ShapeDtypeStruct((1, NG), _f32),
            jax.ShapeDtypeStruct((NG, H), _f32),
        ],
        scratch_shapes=[pltpu.VMEM((8, NG), _f32), pltpu.VMEM((NG, H), _f32)],
    )(e, m, batch, hw)


def _k7_body(ru_ref, s_ref, hhp_ref, ccp_ref, wih_ref, whh_ref,
             bih_ref, bhh_ref, hh_ref, cc_ref):
    r = ru_ref[...] / (jnp.transpose(s_ref[...]) + 1e-16)
    hhp = hhp_ref[...]
    q = jnp.concatenate([hhp, r], axis=1)
    gates = (lax.dot_general(q, wih_ref[...], (((1,), (1,)), ((), ())),
                             preferred_element_type=_f32, precision=lax.Precision.HIGHEST) + bih_ref[...]
             + lax.dot_general(hhp, whh_ref[...], (((1,), (1,)), ((), ())),
                               preferred_element_type=_f32, precision=lax.Precision.HIGHEST) + bhh_ref[...])
    ii, ff, gg, oo = jnp.split(gates, 4, axis=1)
    cc = jax.nn.sigmoid(ff) * ccp_ref[...] + jax.nn.sigmoid(ii) * jnp.tanh(gg)
    hh = jax.nn.sigmoid(oo) * jnp.tanh(cc)
    hh_ref[...] = hh
    cc_ref[...] = cc


def _k7(ru, s, hh_prev, cc_prev, W_ih, W_hh, b_ih, b_hh):
    return pl.pallas_call(
        _k7_body,
        in_specs=[pl.BlockSpec(memory_space=pltpu.VMEM)] * 8,
        out_specs=[pl.BlockSpec(memory_space=pltpu.VMEM)] * 2,
        out_shape=[
            jax.ShapeDtypeStruct((NG, H), _f32),
            jax.ShapeDtypeStruct((NG, H), _f32),
        ],
    )(ru, s, hh_prev, cc_prev, W_ih, W_hh, b_ih, b_hh)


def _k8_body(hw_ref, batch_ref, hh_ref, e_ref, m_ref, acc_ref):
    i = pl.program_id(0)
    oh = _oh(batch_ref[...])
    hb = jnp.dot(oh, hh_ref[...], preferred_element_type=_f32, precision=lax.Precision.HIGHEST)
    e = jnp.sum(hw_ref[...] * hb, axis=1, keepdims=True)
    e_ref[...] = e
    masked = jnp.where(oh > 0, e, _NEG)
    mpart = jnp.max(masked, axis=0)[None]

    @pl.when(i == 0)
    def _():
        acc_ref[...] = jnp.full((8, NG), _NEG, _f32)

    acc_ref[0:1] = jnp.maximum(acc_ref[0:1], mpart)
    @pl.when(i == NBLK - 1)
    def _():
        m_ref[...] = acc_ref[0:1]


def _k8(hw, batch, hh):
    return pl.pallas_call(
        _k8_body,
        grid=(NBLK,),
        in_specs=[
            pl.BlockSpec((RB, H), lambda i: (i, 0)),
            pl.BlockSpec((RB, 1), lambda i: (i, 0)),
            pl.BlockSpec((NG, H), lambda i: (0, 0)),
        ],
        out_specs=[
            pl.BlockSpec((RB, 1), lambda i: (i, 0)),
            pl.BlockSpec((1, NG), lambda i: (0, 0)),
        ],
        out_shape=[
            jax.ShapeDtypeStruct((N, 1), _f32),
            jax.ShapeDtypeStruct((1, NG), _f32),
        ],
        scratch_shapes=[pltpu.VMEM((8, NG), _f32)],
    )(hw, batch, hh)


NPASS = 7


def _kt_body(score_ref, batch_ref, h_ref, m_ref, bih_ref, bhh_ref,
             wih_ref, whh_ref, wm_ref, bm_ref, wo1_ref, bo1_ref,
             wo2_ref, bo2_ref, out_ref, sacc_ref, macc_ref, stacc_ref,
             ruacc_ref):
    i = pl.program_id(0)
    j = lax.rem(i, NBLK)
    oh = _oh(batch_ref[...])
    m = m_ref[...]
    mc = jnp.where(m < -1e29, 0.0, m)
    ex = jnp.exp(score_ref[...] - jnp.sum(oh * mc, axis=1, keepdims=True))
    h5 = h_ref[...]
    bih = bih_ref[...]
    bhh = bhh_ref[...]
    wih = wih_ref[...]
    whh = whh_ref[...]

    gates0 = bih + bhh
    ii0, ff0, gg0, oo0 = jnp.split(gates0, 4, axis=1)
    cc1 = jax.nn.sigmoid(ii0) * jnp.tanh(gg0)
    hh1 = jax.nn.sigmoid(oo0) * jnp.tanh(cc1)

    def lstm(q, hhp, ccp):
        gates = (lax.dot_general(q, wih, (((1,), (1,)), ((), ())),
                                 preferred_element_type=_f32,
                                 precision=lax.Precision.HIGHEST) + bih
                 + lax.dot_general(hhp, whh, (((1,), (1,)), ((), ())),
                                   preferred_element_type=_f32,
                                   precision=lax.Precision.HIGHEST) + bhh)
        ii, ff, gg, oo = jnp.split(gates, 4, axis=1)
        cc = jax.nn.sigmoid(ff) * ccp + jax.nn.sigmoid(ii) * jnp.tanh(gg)
        hh = jax.nn.sigmoid(oo) * jnp.tanh(cc)
        return hh, cc

    def hh_chain(tmax):
        hh = jnp.broadcast_to(hh1, (NG, H))
        cc = jnp.broadcast_to(cc1, (NG, H))
        for t in range(1, tmax):
            ru = ruacc_ref[pl.ds((t - 1) * NG, NG), :]
            st = stacc_ref[t - 1:t]
            r = ru / (jnp.transpose(st) + 1e-16)
            q = jnp.concatenate([hh, r], axis=1)
            hh, cc = lstm(q, hh, cc)
        return hh, cc

    def nw_hw():
        sb = jnp.sum(oh * sacc_ref[0:1], axis=1, keepdims=True)
        nw = ex / (sb + 1e-16)
        return nw, h5 * nw

    def e_of(tmax, nw, hw):
        if tmax == 1:
            return nw * jnp.sum(h5 * hh1, axis=1, keepdims=True)
        hh, _ = hh_chain(tmax)
        return jnp.sum(hw * jnp.dot(oh, hh, preferred_element_type=_f32,
                                    precision=lax.Precision.HIGHEST),
                       axis=1, keepdims=True)

    def accum_max(t, e):
        @pl.when(j == 0)
        def _():
            macc_ref[t - 1:t] = jnp.full((1, NG), _NEG, _f32)
        macc_ref[t - 1:t] = jnp.maximum(
            macc_ref[t - 1:t], jnp.max(jnp.where(oh > 0, e, _NEG),
                                       axis=0)[None])

    def accum_sum_ru(t, e, hw):
        mt = macc_ref[t - 1:t]
        mtc = jnp.where(mt < -1e29, 0.0, mt)
        ext = jnp.exp(e - jnp.sum(oh * mtc, axis=1, keepdims=True))
        spart = jnp.sum(oh * ext, axis=0)[None]
        rupart = lax.dot_general(oh, ext * hw, (((0,), (0,)), ((), ())),
                                 preferred_element_type=_f32,
                                 precision=lax.Precision.HIGHEST)

        @pl.when(j == 0)
        def _():
            stacc_ref[t - 1:t] = jnp.zeros((1, NG), _f32)
            ruacc_ref[pl.ds((t - 1) * NG, NG), :] = jnp.zeros((NG, H), _f32)
        stacc_ref[t - 1:t] += spart
        ruacc_ref[pl.ds((t - 1) * NG, NG), :] += rupart

    @pl.when(i < NBLK)
    def _():
        @pl.when(j == 0)
        def _():
            sacc_ref[...] = jnp.zeros((1, NG), _f32)
        sacc_ref[...] += jnp.sum(oh * ex, axis=0)[None]

    @pl.when((i >= NBLK) & (i < 2 * NBLK))
    def _():
        nw, hw = nw_hw()
        accum_max(1, e_of(1, nw, hw))

    @pl.when((i >= 2 * NBLK) & (i < 3 * NBLK))
    def _():
        nw, hw = nw_hw()
        accum_sum_ru(1, e_of(1, nw, hw), hw)

    @pl.when((i >= 3 * NBLK) & (i < 4 * NBLK))
    def _():
        nw, hw = nw_hw()
        accum_max(2, e_of(2, nw, hw))

    @pl.when((i >= 4 * NBLK) & (i < 5 * NBLK))
    def _():
        nw, hw = nw_hw()
        accum_sum_ru(2, e_of(2, nw, hw), hw)

    @pl.when((i >= 5 * NBLK) & (i < 6 * NBLK))
    def _():
        nw, hw = nw_hw()
        accum_max(3, e_of(3, nw, hw))

    @pl.when(i >= 6 * NBLK)
    def _():
        nw, hw = nw_hw()
        accum_sum_ru(3, e_of(3, nw, hw), hw)

        @pl.when(i == NPASS * NBLK - 1)
        def _():
            hh3, _ = hh_chain(3)
            ru3 = ruacc_ref[pl.ds(2 * NG, NG), :]
            st3 = stacc_ref[2:3]
            r3 = ru3 / (jnp.transpose(st3) + 1e-16)
            q3 = jnp.concatenate([hh3, r3], axis=1)
            mm = jnp.maximum(
                jnp.dot(q3, wm_ref[...], preferred_element_type=_f32,
                        precision=lax.Precision.HIGHEST) + bm_ref[...], 0.0)
            mm = jnp.maximum(
                jnp.dot(mm, wo1_ref[...], preferred_element_type=_f32,
                        precision=lax.Precision.HIGHEST) + bo1_ref[...], 0.0)
            mm = jnp.dot(mm, wo2_ref[...], preferred_element_type=_f32,
                         precision=lax.Precision.HIGHEST) + bo2_ref[...]
            out_ref[...] = mm


def _kt(score, m, batch, h5, b_ih, b_hh, W_ih, W_hh,
        W_m, b_m, W_o1, b_o1, W_o2, b_o2):
    pj = lambda i: (lax.rem(i, NBLK), 0)
    c0 = lambda i: (0, 0)
    return pl.pallas_call(
        _kt_body,
        grid=(NPASS * NBLK,),
        in_specs=[
            pl.BlockSpec((RB, 1), pj),
            pl.BlockSpec((RB, 1), pj),
            pl.BlockSpec((RB, H), pj),
            pl.BlockSpec((1, NG), c0),
            pl.BlockSpec((1, 4 * H), c0),
            pl.BlockSpec((1, 4 * H), c0),
            pl.BlockSpec((4 * H, 2 * H), c0),
            pl.BlockSpec((4 * H, H), c0),
            pl.BlockSpec((2 * H, 2 * H), c0),
            pl.BlockSpec((1, 2 * H), c0),
            pl.BlockSpec((2 * H, 32), c0),
            pl.BlockSpec((1, 32), c0),
            pl.BlockSpec((32, 1), c0),
            pl.BlockSpec((1, 1), c0),
        ],
        out_specs=pl.BlockSpec((NG, 1), c0),
        out_shape=jax.ShapeDtypeStruct((NG, 1), _f32),
        scratch_shapes=[
            pltpu.VMEM((1, NG), _f32),
            pltpu.VMEM((8, NG), _f32),
            pltpu.VMEM((8, NG), _f32),
            pltpu.VMEM((3 * NG, H), _f32),
        ],
    )(score, batch, h5, m, b_ih, b_hh, W_ih, W_hh,
      W_m, b_m, W_o1, b_o1, W_o2, b_o2)


def _k13_body(ru_ref, s_ref, hhp_ref, wm_ref, bm_ref, wo1_ref, bo1_ref,
              wo2_ref, bo2_ref, out_ref):
    r = ru_ref[...] / (jnp.transpose(s_ref[...]) + 1e-16)
    hhp = hhp_ref[...]
    q = jnp.concatenate([hhp, r], axis=1)
    mm = jnp.maximum(jnp.dot(q, wm_ref[...],
                             preferred_element_type=_f32, precision=lax.Precision.HIGHEST) + bm_ref[...], 0.0)
    mm = jnp.maximum(jnp.dot(mm, wo1_ref[...],
                             preferred_element_type=_f32, precision=lax.Precision.HIGHEST) + bo1_ref[...], 0.0)
    mm = jnp.dot(mm, wo2_ref[...], preferred_element_type=_f32, precision=lax.Precision.HIGHEST) + bo2_ref[...]
    out_ref[...] = mm


def _k13(ru, s, hh_prev, W_m, b_m, W_o1, b_o1, W_o2, b_o2):
    return pl.pallas_call(
        _k13_body,
        in_specs=[pl.BlockSpec(memory_space=pltpu.VMEM)] * 9,
        out_specs=pl.BlockSpec(memory_space=pltpu.VMEM),
        out_shape=jax.ShapeDtypeStruct((NG, 1), _f32),
    )(ru, s, hh_prev, W_m, b_m, W_o1, b_o1, W_o2, b_o2)


E_PAD = 802816
EW = E_PAD // 32
CW = 1568
KE = 256
ET = E_PAD // 16
NA = 50176
NSL = NA // 16
ZB = 64
NP = 51200

_mesh = plsc.VectorSubcoreMesh(core_axis_name="c", subcore_axis_name="s")


def _zero16(buf, nrows, width):
    def z(i, _):
        for w in range(width // 16):
            buf[i, pl.ds(w * 16, 16)] = jnp.zeros((16,), _f32)
        return 0
    lax.fori_loop(0, nrows, z, 0)


def _deg_body(col_hbm, ew_hbm, out_hbm, degbuf, colb, ewb):
    cid = lax.axis_index("c")
    sid = lax.axis_index("s")
    wid = sid * 2 + cid

    def z(i, _):
        degbuf[pl.ds(i * 16, 16)] = jnp.zeros((16,), _f32)
        return 0
    lax.fori_loop(0, N // 16, z, 0)

    base = wid * EW
    lanes = lax.iota(jnp.int32, 16)
    for ch in range(16):
        pltpu.sync_copy(col_hbm.at[pl.ds(base + ch * CW, CW)], colb)
        pltpu.sync_copy(ew_hbm.at[pl.ds(base + ch * CW, CW)], ewb)

        def acc(i, _):
            cv = colb[pl.ds(i * 16, 16)]
            wv = ewb[pl.ds(i * 16, 16)]
            for k in range(16):
                plsc.addupdate_scatter(degbuf, [cv], wv, mask=lanes == k)
            return 0
        lax.fori_loop(0, CW // 16, acc, 0)
    pltpu.sync_copy(degbuf, out_hbm.at[wid, pl.ds(0, N)])


def _sc_deg(colp, ewp):
    f = pl.kernel(
        _deg_body,
        out_type=jax.ShapeDtypeStruct((32, NP), _f32),
        mesh=_mesh,
        compiler_params=pltpu.CompilerParams(use_tc_tiling_on_sc=False, needs_layout_passes=False),
        scratch_types=[
            pltpu.VMEM((N,), _f32),
            pltpu.VMEM((CW,), jnp.int32),
            pltpu.VMEM((CW,), _f32),
        ],
    )
    return f(colp, ewp)


def _norm_body(dis_hbm, row_hbm, col_hbm, ew_hbm, norm_hbm,
               disbuf, rowb, colb, ewb, nb):
    cid = lax.axis_index("c")
    sid = lax.axis_index("s")
    wid = sid * 2 + cid
    pltpu.sync_copy(dis_hbm, disbuf)
    base = wid * EW
    for ch in range(16):
        off = base + ch * CW
        pltpu.sync_copy(row_hbm.at[pl.ds(off, CW)], rowb)
        pltpu.sync_copy(col_hbm.at[pl.ds(off, CW)], colb)
        pltpu.sync_copy(ew_hbm.at[pl.ds(off, CW)], ewb)

        def grp(i, _):
            rv = rowb[pl.ds(i * 16, 16)]
            cv = colb[pl.ds(i * 16, 16)]
            dr = plsc.load_gather(disbuf, [rv])
            dc = plsc.load_gather(disbuf, [cv])
            nb[pl.ds(i * 16, 16)] = dr * ewb[pl.ds(i * 16, 16)] * dc
            return 0
        lax.fori_loop(0, CW // 16, grp, 0)
        pltpu.sync_copy(nb, norm_hbm.at[pl.ds(off, CW)])


def _sc_norm(dis_flat, rowp, colp, ewp):
    f = pl.kernel(
        _norm_body,
        out_type=jax.ShapeDtypeStruct((E_PAD,), _f32),
        mesh=_mesh,
        compiler_params=pltpu.CompilerParams(use_tc_tiling_on_sc=False, needs_layout_passes=False),
        scratch_types=[
            pltpu.VMEM((N,), _f32),
            pltpu.VMEM((CW,), jnp.int32),
            pltpu.VMEM((CW,), jnp.int32),
            pltpu.VMEM((CW,), _f32),
            pltpu.VMEM((CW,), _f32),
        ],
    )
    return f(dis_flat, rowp, colp, ewp)


NCH = ET // KE


def _scatter_body(xw_hbm, row_hbm, col_hbm, norm_hbm, out_hbm,
                  rowb0, colb0, normb0, rowb1, colb1, normb1,
                  gbuf0, gbuf1, sbuf, zbuf, acc,
                  isem0, isem1, gsem0, gsem1):
    cid = lax.axis_index("c")
    sid = lax.axis_index("s")
    rowb = (rowb0, rowb1)
    colb = (colb0, colb1)
    normb = (normb0, normb1)
    gbuf = (gbuf0, gbuf1)
    isem = (isem0, isem1)
    gsem = (gsem0, gsem1)
    base = sid * ET

    def start_idx(c, p):
        off = base + c * KE
        pltpu.async_copy(row_hbm.at[pl.ds(off, KE)], rowb[p], isem[p])
        pltpu.async_copy(col_hbm.at[pl.ds(off, KE)], colb[p], isem[p])
        pltpu.async_copy(norm_hbm.at[pl.ds(off, KE)], normb[p], isem[p])

    def drain_idx(p):
        pltpu.make_async_copy(row_hbm.at[pl.ds(0, KE)], rowb[p], isem[p]).wait()
        pltpu.make_async_copy(col_hbm.at[pl.ds(0, KE)], colb[p], isem[p]).wait()
        pltpu.make_async_copy(norm_hbm.at[pl.ds(0, KE)], normb[p], isem[p]).wait()

    def start_gather(p):
        pltpu.async_copy(xw_hbm.at[cid].at[rowb[p]], gbuf[p], gsem[p])

    def drain_gather(p):
        pltpu.make_async_copy(xw_hbm.at[cid].at[rowb[p]], gbuf[p],
                              gsem[p]).wait()

    def process(p):
        def scale(g, _):
            nvec = normb[p][pl.ds(g * 16, 16)]
            for k in range(16):
                i = g * 16 + k
                nv = jnp.full((16,), nvec[k], _f32)
                sbuf[i, pl.ds(0, 16)] = gbuf[p][i, pl.ds(0, 16)] * nv
                sbuf[i, pl.ds(16, 16)] = gbuf[p][i, pl.ds(16, 16)] * nv
            return 0
        lax.fori_loop(0, KE // 16, scale, 0)
        pltpu.sync_copy(sbuf, acc.at[colb[p]], add=True)

    _zero16(zbuf, ZB, 32)
    for z in range(NSL // ZB):
        pltpu.sync_copy(zbuf, acc.at[pl.ds(sid * NSL + z * ZB, ZB)])
    plsc.subcore_barrier()

    start_idx(0, 0)
    drain_idx(0)
    start_gather(0)
    start_idx(1, 1)

    def dchunk(jj, _):
        for p in (0, 1):
            c = jj * 2 + p
            q = 1 - p

            @pl.when(c + 1 < NCH)
            def _():
                drain_idx(q)
                start_gather(q)
            drain_gather(p)
            process(p)

            @pl.when(c + 2 < NCH)
            def _():
                start_idx(c + 2, p)
        return 0
    lax.fori_loop(0, NCH // 2, dchunk, 0)
    plsc.subcore_barrier()
    pltpu.sync_copy(acc.at[pl.ds(sid * NSL, NSL)],
                    out_hbm.at[pl.ds(sid * NSL, NSL), pl.ds(cid * 32, 32)])


def _sc_scatter(xw2, rowp, colp, normp):
    f = pl.kernel(
        _scatter_body,
        out_type=jax.ShapeDtypeStruct((NA, H), _f32),
        mesh=_mesh,
        compiler_params=pltpu.CompilerParams(use_tc_tiling_on_sc=False, needs_layout_passes=False),
        scratch_types=[
            pltpu.VMEM((KE,), jnp.int32),
            pltpu.VMEM((KE,), jnp.int32),
            pltpu.VMEM((KE,), _f32),
            pltpu.VMEM((KE,), jnp.int32),
            pltpu.VMEM((KE,), jnp.int32),
            pltpu.VMEM((KE,), _f32),
            pltpu.VMEM((KE, 32), _f32),
            pltpu.VMEM((KE, 32), _f32),
            pltpu.VMEM((KE, 32), _f32),
            pltpu.VMEM((ZB, 32), _f32),
            pltpu.VMEM_SHARED((NA, 32), _f32),
            pltpu.SemaphoreType.DMA,
            pltpu.SemaphoreType.DMA,
            pltpu.SemaphoreType.DMA,
            pltpu.SemaphoreType.DMA,
        ],
    )
    return f(xw2, rowp, colp, normp)


def _k0_body(p_ref, deg_ref):
    deg_ref[...] = jnp.sum(p_ref[...], axis=0) + 2.0


def _k0(partials3):
    return pl.pallas_call(
        _k0_body,
        grid=(10,),
        in_specs=[pl.BlockSpec((32, 40, 128), lambda i: (0, i, 0))],
        out_specs=pl.BlockSpec((40, 128), lambda i: (i, 0)),
        out_shape=jax.ShapeDtypeStruct((400, 128), _f32),
    )(partials3)


def kernel(x, edge_index, edge_dist, global_info, batch,
           W_in, b_in, W_conv, b_conv, bn_g, bn_b, W_gin, b_gin,
           W_gls, b_gls, W_gout, b_gout, W_ih, W_hh, b_ih, b_hh,
           W_m, b_m, W_o1, b_o1, W_o2, b_o2):
    row, col = edge_index[0], edge_index[1]
    b_in2 = b_in.reshape(1, H)
    bn_g2 = bn_g.reshape(1, H)
    bn_b2 = bn_b.reshape(1, H)
    batch2 = batch.reshape(N, 1)
    b_ih2 = b_ih.reshape(1, 4 * H)
    b_hh2 = b_hh.reshape(1, 4 * H)

    padi = jnp.zeros((E_PAD - E,), jnp.int32)
    rowp = jnp.concatenate([row, padi])
    colp = jnp.concatenate([col, padi])
    ewp = jnp.concatenate([edge_dist, jnp.zeros((E_PAD - E,), _f32)])

    deg_part = _sc_deg(colp, ewp)
    deg = _k0(deg_part.reshape(32, 400, 128)).reshape(NP, 1)
    h0, xw, dis, sn = _k1(x, W_in, b_in2, W_conv[0], deg)
    norm = _sc_norm(dis.reshape(N), rowp, colp, ewp)

    prev = h0
    for i in range(5):
        scat = _sc_scatter(xw, rowp, colp, norm)
        if i < 4:
            prev, xw = _k23(scat, sn, b_conv[i].reshape(1, H),
                            prev, bn_g2, bn_b2, W_conv[i], W_conv[i + 1])
        else:
            h5, score, m = _k23l(
                scat, sn, b_conv[i].reshape(1, H), prev, bn_g2, bn_b2,
                W_conv[i], global_info, batch2,
                W_gin[:H], W_gin[H:], b_gin.reshape(1, H),
                W_gls[0], b_gls[0].reshape(1, H),
                W_gls[1], b_gls[1].reshape(1, H),
                W_gout, b_gout.reshape(1, 1))

    out = _kt(score, m, batch2, h5, b_ih2, b_hh2, W_ih, W_hh,
              W_m, b_m.reshape(1, 2 * H), W_o1, b_o1.reshape(1, 32),
              W_o2, b_o2.reshape(1, 1))
    return out.reshape(1, NG)


# speedup vs baseline: 1.6400x; 1.6400x over previous
import functools

import jax
import jax.numpy as jnp
from jax import lax
from jax.experimental import pallas as pl
from jax.experimental.pallas import tpu as pltpu
from jax.experimental.pallas import tpu_sc as plsc

N = 50000
E = 800000
IN = 128
H = 64
NG = 64
G = 107

RB = 5000
NBLK = N // RB

_f32 = jnp.float32
_NEG = -1e30


def _oh(batch_blk):
    segs = lax.broadcasted_iota(jnp.int32, (1, NG), 1)
    return (batch_blk == segs).astype(_f32)


def _k1_body(x_ref, win_ref, bin_ref, w0_ref, deg_ref,
             h_ref, xw_ref, dis_ref, sn_ref):
    h = jnp.maximum(jnp.dot(x_ref[...], win_ref[...],
                            preferred_element_type=_f32, precision=lax.Precision.HIGHEST) + bin_ref[...], 0.0)
    h_ref[...] = h
    xw = jnp.dot(h, w0_ref[...], preferred_element_type=_f32, precision=lax.Precision.HIGHEST)
    xw_ref[0] = xw[:, :32]
    xw_ref[1] = xw[:, 32:]
    deg = deg_ref[...]
    dis = jnp.where(deg > 0, lax.rsqrt(jnp.where(deg > 0, deg, 1.0)), 0.0)
    dis_ref[...] = dis
    sn_ref[...] = 2.0 * dis * dis


def _k1(x, W_in, b_in, W0, deg):
    return pl.pallas_call(
        _k1_body,
        grid=(NBLK,),
        in_specs=[
            pl.BlockSpec((RB, IN), lambda i: (i, 0)),
            pl.BlockSpec((IN, H), lambda i: (0, 0)),
            pl.BlockSpec((1, H), lambda i: (0, 0)),
            pl.BlockSpec((H, H), lambda i: (0, 0)),
            pl.BlockSpec((RB, 1), lambda i: (i, 0)),
        ],
        out_specs=[
            pl.BlockSpec((RB, H), lambda i: (i, 0)),
            pl.BlockSpec((2, RB, 32), lambda i: (0, i, 0)),
            pl.BlockSpec((RB, 1), lambda i: (i, 0)),
            pl.BlockSpec((RB, 1), lambda i: (i, 0)),
        ],
        out_shape=[
            jax.ShapeDtypeStruct((N, H), _f32),
            jax.ShapeDtypeStruct((2, N, 32), _f32),
            jax.ShapeDtypeStruct((N, 1), _f32),
            jax.ShapeDtypeStruct((N, 1), _f32),
        ],
    )(x, W_in, b_in, W0, deg)


def _k2_body(scat_ref, xw_ref, sn_ref, b_ref, out_ref, sums_ref, acc_ref):
    i = pl.program_id(0)
    scat = jnp.concatenate([scat_ref[0], scat_ref[1]], axis=1)
    xw = jnp.concatenate([xw_ref[0], xw_ref[1]], axis=1)
    out = scat + sn_ref[...] * xw + b_ref[...]
    out_ref[...] = out
    ps = jnp.sum(out, axis=0)[None]
    ps2 = jnp.sum(out * out, axis=0)[None]
    part = jnp.concatenate([ps, ps2, jnp.zeros((6, H), _f32)], axis=0)

    @pl.when(i == 0)
    def _():
        acc_ref[...] = jnp.zeros((8, H), _f32)

    acc_ref[...] += part
    @pl.when(i == NBLK - 1)
    def _():
        sums_ref[...] = acc_ref[...]


def _k2(scat, xw, sn, b_conv_i):
    return pl.pallas_call(
        _k2_body,
        grid=(NBLK,),
        in_specs=[
            pl.BlockSpec((2, RB, 32), lambda i: (0, i, 0)),
            pl.BlockSpec((2, RB, 32), lambda i: (0, i, 0)),
            pl.BlockSpec((RB, 1), lambda i: (i, 0)),
            pl.BlockSpec((1, H), lambda i: (0, 0)),
        ],
        out_specs=[
            pl.BlockSpec((RB, H), lambda i: (i, 0)),
            pl.BlockSpec((8, H), lambda i: (0, 0)),
        ],
        out_shape=[
            jax.ShapeDtypeStruct((N, H), _f32),
            jax.ShapeDtypeStruct((8, H), _f32),
        ],
        scratch_shapes=[pltpu.VMEM((8, H), _f32)],
    )(scat, xw, sn, b_conv_i)


def _k23_body(scat_ref, sn_ref, b_ref, prev_ref, g_ref, bb_ref,
              wi_ref, wn_ref, h_ref, xwn_ref, acc_ref):
    i = pl.program_id(0)
    prev = prev_ref[...]
    xw = jnp.dot(prev, wi_ref[...], preferred_element_type=_f32,
                 precision=lax.Precision.HIGHEST)
    out = scat_ref[...] + sn_ref[...] * xw + b_ref[...]

    @pl.when(i < NBLK)
    def _():
        ps = jnp.sum(out, axis=0)[None]
        ps2 = jnp.sum(out * out, axis=0)[None]
        part = jnp.concatenate([ps, ps2, jnp.zeros((6, H), _f32)], axis=0)

        @pl.when(i == 0)
        def _():
            acc_ref[...] = jnp.zeros((8, H), _f32)

        acc_ref[...] += part

    @pl.when(i >= NBLK)
    def _():
        sums = acc_ref[...]
        m = sums[0:1] * (1.0 / N)
        v = sums[1:2] * (1.0 / N) - m * m
        inv = lax.rsqrt(v + 1e-5)
        hn = (out - m) * inv * g_ref[...] + bb_ref[...] + prev
        h_ref[...] = hn
        xwn = jnp.dot(hn, wn_ref[...], preferred_element_type=_f32,
                      precision=lax.Precision.HIGHEST)
        xwn_ref[0] = xwn[:, :32]
        xwn_ref[1] = xwn[:, 32:]


def _k23(scat, sn, b_conv_i, prev, bn_g, bn_b, W_i, W_next):
    pj = lambda i: (lax.rem(i, NBLK), 0)
    p2 = lambda i: (jnp.maximum(i - NBLK, 0), 0)
    c0 = lambda i: (0, 0)
    return pl.pallas_call(
        _k23_body,
        grid=(2 * NBLK,),
        in_specs=[
            pl.BlockSpec((RB, H), pj),
            pl.BlockSpec((RB, 1), pj),
            pl.BlockSpec((1, H), c0),
            pl.BlockSpec((RB, H), pj),
            pl.BlockSpec((1, H), c0),
            pl.BlockSpec((1, H), c0),
            pl.BlockSpec((H, H), c0),
            pl.BlockSpec((H, H), c0),
        ],
        out_specs=[
            pl.BlockSpec((RB, H), p2),
            pl.BlockSpec((2, RB, 32), lambda i: (0, jnp.maximum(i - NBLK, 0),
                                                 0)),
        ],
        out_shape=[
            jax.ShapeDtypeStruct((N, H), _f32),
            jax.ShapeDtypeStruct((2, N, 32), _f32),
        ],
        scratch_shapes=[pltpu.VMEM((8, H), _f32)],
    )(scat, sn, b_conv_i, prev, bn_g, bn_b, W_i, W_next)


def _k3_body(out_ref, prev_ref, sums_ref, g_ref, bb_ref, wn_ref,
             h_ref, xw_ref):
    sums = sums_ref[...]
    m = sums[0:1] * (1.0 / N)
    v = sums[1:2] * (1.0 / N) - m * m
    inv = lax.rsqrt(v + 1e-5)
    hn = (out_ref[...] - m) * inv * g_ref[...] + bb_ref[...] + prev_ref[...]
    h_ref[...] = hn
    xw = jnp.dot(hn, wn_ref[...], preferred_element_type=_f32, precision=lax.Precision.HIGHEST)
    xw_ref[0] = xw[:, :32]
    xw_ref[1] = xw[:, 32:]


def _k3(out, prev, sums, bn_g, bn_b, W_next):
    return pl.pallas_call(
        _k3_body,
        grid=(NBLK,),
        in_specs=[
            pl.BlockSpec((RB, H), lambda i: (i, 0)),
            pl.BlockSpec((RB, H), lambda i: (i, 0)),
            pl.BlockSpec((8, H), lambda i: (0, 0)),
            pl.BlockSpec((1, H), lambda i: (0, 0)),
            pl.BlockSpec((1, H), lambda i: (0, 0)),
            pl.BlockSpec((H, H), lambda i: (0, 0)),
        ],
        out_specs=[
            pl.BlockSpec((RB, H), lambda i: (i, 0)),
            pl.BlockSpec((2, RB, 32), lambda i: (0, i, 0)),
        ],
        out_shape=[
            jax.ShapeDtypeStruct((N, H), _f32),
            jax.ShapeDtypeStruct((2, N, 32), _f32),
        ],
    )(out, prev, sums, bn_g, bn_b, W_next)


def _k23l_body(scat_ref, sn_ref, b_ref, prev_ref, g_ref, bb_ref, wi_ref,
               gi_ref, batch_ref, wgh_ref, wgg_ref, bgin_ref, wg0_ref,
               bg0_ref, wg1_ref, bg1_ref, wgo_ref, bgo_ref,
               h_ref, score_ref, m_ref, acc_ref, accm_ref):
    i = pl.program_id(0)
    prev = prev_ref[...]
    xw = jnp.dot(prev, wi_ref[...], preferred_element_type=_f32,
                 precision=lax.Precision.HIGHEST)
    out = scat_ref[...] + sn_ref[...] * xw + b_ref[...]

    @pl.when(i < NBLK)
    def _():
        ps = jnp.sum(out, axis=0)[None]
        ps2 = jnp.sum(out * out, axis=0)[None]
        part = jnp.concatenate([ps, ps2, jnp.zeros((6, H), _f32)], axis=0)

        @pl.when(i == 0)
        def _():
            acc_ref[...] = jnp.zeros((8, H), _f32)

        acc_ref[...] += part

    @pl.when(i >= NBLK)
    def _():
        j = i - NBLK
        sums = acc_ref[...]
        m = sums[0:1] * (1.0 / N)
        v = sums[1:2] * (1.0 / N) - m * m
        inv = lax.rsqrt(v + 1e-5)
        hn = (out - m) * inv * g_ref[...] + bb_ref[...] + prev
        h_ref[...] = hn
        g1 = jnp.maximum(
            jnp.dot(hn, wgh_ref[...], preferred_element_type=_f32,
                    precision=lax.Precision.HIGHEST)
            + jnp.dot(gi_ref[...], wgg_ref[...], preferred_element_type=_f32,
                      precision=lax.Precision.HIGHEST)
            + bgin_ref[...], 0.0)
        g2 = jnp.maximum(
            jnp.dot(g1, wg0_ref[...], preferred_element_type=_f32,
                    precision=lax.Precision.HIGHEST) + bg0_ref[...], 0.0)
        g3 = jnp.maximum(
            jnp.dot(g2, wg1_ref[...], preferred_element_type=_f32,
                    precision=lax.Precision.HIGHEST) + bg1_ref[...], 0.0)
        score = jnp.dot(g3, wgo_ref[...], preferred_element_type=_f32,
                        precision=lax.Precision.HIGHEST) + bgo_ref[...]
        score_ref[...] = score
        oh = _oh(batch_ref[...])
        masked = jnp.where(oh > 0, score, _NEG)
        mpart = jnp.max(masked, axis=0)[None]

        @pl.when(j == 0)
        def _():
            accm_ref[...] = jnp.full((8, NG), _NEG, _f32)

        accm_ref[0:1] = jnp.maximum(accm_ref[0:1], mpart)
        @pl.when(j == NBLK - 1)
        def _():
            m_ref[...] = accm_ref[0:1]


def _k23l(scat, sn, b_conv_i, prev, bn_g, bn_b, W_i, gi, batch,
          Wg_h, Wg_g, b_gin, Wg0, bg0, Wg1, bg1, Wgo, bgo):
    pj = lambda i: (lax.rem(i, NBLK), 0)
    p2 = lambda i: (jnp.maximum(i - NBLK, 0), 0)
    c0 = lambda i: (0, 0)
    return pl.pallas_call(
        _k23l_body,
        grid=(2 * NBLK,),
        in_specs=[
            pl.BlockSpec((RB, H), pj),
            pl.BlockSpec((RB, 1), pj),
            pl.BlockSpec((1, H), c0),
            pl.BlockSpec((RB, H), pj),
            pl.BlockSpec((1, H), c0),
            pl.BlockSpec((1, H), c0),
            pl.BlockSpec((H, H), c0),
            pl.BlockSpec((RB, G), p2),
            pl.BlockSpec((RB, 1), p2),
            pl.BlockSpec((H, H), c0),
            pl.BlockSpec((G, H), c0),
            pl.BlockSpec((1, H), c0),
            pl.BlockSpec((H, H), c0),
            pl.BlockSpec((1, H), c0),
            pl.BlockSpec((H, H), c0),
            pl.BlockSpec((1, H), c0),
            pl.BlockSpec((H, 1), c0),
            pl.BlockSpec((1, 1), c0),
        ],
        out_specs=[
            pl.BlockSpec((RB, H), p2),
            pl.BlockSpec((RB, 1), p2),
            pl.BlockSpec((1, NG), c0),
        ],
        out_shape=[
            jax.ShapeDtypeStruct((N, H), _f32),
            jax.ShapeDtypeStruct((N, 1), _f32),
            jax.ShapeDtypeStruct((1, NG), _f32),
        ],
        scratch_shapes=[pltpu.VMEM((8, H), _f32), pltpu.VMEM((8, NG), _f32)],
    )(scat, sn, b_conv_i, prev, bn_g, bn_b, W_i, gi, batch,
      Wg_h, Wg_g, b_gin, Wg0, bg0, Wg1, bg1, Wgo, bgo)


def _k3l_body(out_ref, prev_ref, sums_ref, g_ref, bb_ref, gi_ref, batch_ref,
              wgh_ref, wgg_ref, bgin_ref, wg0_ref, bg0_ref, wg1_ref, bg1_ref,
              wgo_ref, bgo_ref,
              h_ref, score_ref, m_ref, acc_ref):
    i = pl.program_id(0)
    sums = sums_ref[...]
    m = sums[0:1] * (1.0 / N)
    v = sums[1:2] * (1.0 / N) - m * m
    inv = lax.rsqrt(v + 1e-5)
    hn = (out_ref[...] - m) * inv * g_ref[...] + bb_ref[...] + prev_ref[...]
    h_ref[...] = hn
    g1 = jnp.maximum(
        jnp.dot(hn, wgh_ref[...], preferred_element_type=_f32, precision=lax.Precision.HIGHEST)
        + jnp.dot(gi_ref[...], wgg_ref[...], preferred_element_type=_f32, precision=lax.Precision.HIGHEST)
        + bgin_ref[...], 0.0)
    g2 = jnp.maximum(
        jnp.dot(g1, wg0_ref[...], preferred_element_type=_f32, precision=lax.Precision.HIGHEST) + bg0_ref[...],
        0.0)
    g3 = jnp.maximum(
        jnp.dot(g2, wg1_ref[...], preferred_element_type=_f32, precision=lax.Precision.HIGHEST) + bg1_ref[...],
        0.0)
    score = jnp.dot(g3, wgo_ref[...], preferred_element_type=_f32, precision=lax.Precision.HIGHEST) \
        + bgo_ref[...]
    score_ref[...] = score
    oh = _oh(batch_ref[...])
    masked = jnp.where(oh > 0, score, _NEG)
    mpart = jnp.max(masked, axis=0)[None]

    @pl.when(i == 0)
    def _():
        acc_ref[...] = jnp.full((8, NG), _NEG, _f32)

    acc_ref[0:1] = jnp.maximum(acc_ref[0:1], mpart)
    @pl.when(i == NBLK - 1)
    def _():
        m_ref[...] = acc_ref[0:1]


def _k3l(out, prev, sums, bn_g, bn_b, gi, batch, Wg_h, Wg_g, b_gin,
         Wg0, bg0, Wg1, bg1, Wgo, bgo):
    return pl.pallas_call(
        _k3l_body,
        grid=(NBLK,),
        in_specs=[
            pl.BlockSpec((RB, H), lambda i: (i, 0)),
            pl.BlockSpec((RB, H), lambda i: (i, 0)),
            pl.BlockSpec((8, H), lambda i: (0, 0)),
            pl.BlockSpec((1, H), lambda i: (0, 0)),
            pl.BlockSpec((1, H), lambda i: (0, 0)),
            pl.BlockSpec((RB, G), lambda i: (i, 0)),
            pl.BlockSpec((RB, 1), lambda i: (i, 0)),
            pl.BlockSpec((H, H), lambda i: (0, 0)),
            pl.BlockSpec((G, H), lambda i: (0, 0)),
            pl.BlockSpec((1, H), lambda i: (0, 0)),
            pl.BlockSpec((H, H), lambda i: (0, 0)),
            pl.BlockSpec((1, H), lambda i: (0, 0)),
            pl.BlockSpec((H, H), lambda i: (0, 0)),
            pl.BlockSpec((1, H), lambda i: (0, 0)),
            pl.BlockSpec((H, 1), lambda i: (0, 0)),
            pl.BlockSpec((1, 1), lambda i: (0, 0)),
        ],
        out_specs=[
            pl.BlockSpec((RB, H), lambda i: (i, 0)),
            pl.BlockSpec((RB, 1), lambda i: (i, 0)),
            pl.BlockSpec((1, NG), lambda i: (0, 0)),
        ],
        out_shape=[
            jax.ShapeDtypeStruct((N, H), _f32),
            jax.ShapeDtypeStruct((N, 1), _f32),
            jax.ShapeDtypeStruct((1, NG), _f32),
        ],
        scratch_shapes=[pltpu.VMEM((8, NG), _f32)],
    )(out, prev, sums, bn_g, bn_b, gi, batch, Wg_h, Wg_g, b_gin,
      Wg0, bg0, Wg1, bg1, Wgo, bgo)


def _k4_body(score_ref, batch_ref, m_ref, h_ref, bih_ref, bhh_ref,
             ex_ref, s_ref, d1_ref, acc_ref):
    i = pl.program_id(0)
    oh = _oh(batch_ref[...])
    m = m_ref[...]
    mc = jnp.where(m < -1e29, 0.0, m)
    mb = jnp.sum(oh * mc, axis=1, keepdims=True)
    ex = jnp.exp(score_ref[...] - mb)
    ex_ref[...] = ex
    spart = jnp.sum(oh * ex, axis=0)[None]

    gates = bih_ref[...] + bhh_ref[...]
    ii, ff, gg, oo = jnp.split(gates, 4, axis=1)
    cc1 = jax.nn.sigmoid(ii) * jnp.tanh(gg)
    hh1 = jax.nn.sigmoid(oo) * jnp.tanh(cc1)
    d1_ref[...] = jnp.sum(h_ref[...] * hh1, axis=1, keepdims=True)

    @pl.when(i == 0)
    def _():
        acc_ref[...] = jnp.zeros((8, NG), _f32)

    acc_ref[0:1] += spart
    @pl.when(i == NBLK - 1)
    def _():
        s_ref[...] = acc_ref[0:1]


def _k4(score, batch, m, h5, b_ih, b_hh):
    return pl.pallas_call(
        _k4_body,
        grid=(NBLK,),
        in_specs=[
            pl.BlockSpec((RB, 1), lambda i: (i, 0)),
            pl.BlockSpec((RB, 1), lambda i: (i, 0)),
            pl.BlockSpec((1, NG), lambda i: (0, 0)),
            pl.BlockSpec((RB, H), lambda i: (i, 0)),
            pl.BlockSpec((1, 4 * H), lambda i: (0, 0)),
            pl.BlockSpec((1, 4 * H), lambda i: (0, 0)),
        ],
        out_specs=[
            pl.BlockSpec((RB, 1), lambda i: (i, 0)),
            pl.BlockSpec((1, NG), lambda i: (0, 0)),
            pl.BlockSpec((RB, 1), lambda i: (i, 0)),
        ],
        out_shape=[
            jax.ShapeDtypeStruct((N, 1), _f32),
            jax.ShapeDtypeStruct((1, NG), _f32),
            jax.ShapeDtypeStruct((N, 1), _f32),
        ],
        scratch_shapes=[pltpu.VMEM((8, NG), _f32)],
    )(score, batch, m, h5, b_ih, b_hh)


def _k5_body(ex_ref, s_ref, batch_ref, h_ref, d1_ref,
             hw_ref, e1_ref, m1_ref, acc_ref):
    i = pl.program_id(0)
    oh = _oh(batch_ref[...])
    sb = jnp.sum(oh * s_ref[...], axis=1, keepdims=True)
    nw = ex_ref[...] / (sb + 1e-16)
    hw = h_ref[...] * nw
    hw_ref[...] = hw
    e1 = nw * d1_ref[...]
    e1_ref[...] = e1
    masked = jnp.where(oh > 0, e1, _NEG)
    mpart = jnp.max(masked, axis=0)[None]

    @pl.when(i == 0)
    def _():
        acc_ref[...] = jnp.full((8, NG), _NEG, _f32)

    acc_ref[0:1] = jnp.maximum(acc_ref[0:1], mpart)
    @pl.when(i == NBLK - 1)
    def _():
        m1_ref[...] = acc_ref[0:1]


def _k5(ex, s, batch, h5, d1):
    return pl.pallas_call(
        _k5_body,
        grid=(NBLK,),
        in_specs=[
            pl.BlockSpec((RB, 1), lambda i: (i, 0)),
            pl.BlockSpec((1, NG), lambda i: (0, 0)),
            pl.BlockSpec((RB, 1), lambda i: (i, 0)),
            pl.BlockSpec((RB, H), lambda i: (i, 0)),
            pl.BlockSpec((RB, 1), lambda i: (i, 0)),
        ],
        out_specs=[
            pl.BlockSpec((RB, H), lambda i: (i, 0)),
            pl.BlockSpec((RB, 1), lambda i: (i, 0)),
            pl.BlockSpec((1, NG), lambda i: (0, 0)),
        ],
        out_shape=[
            jax.ShapeDtypeStruct((N, H), _f32),
            jax.ShapeDtypeStruct((N, 1), _f32),
            jax.ShapeDtypeStruct((1, NG), _f32),
        ],
        scratch_shapes=[pltpu.VMEM((8, NG), _f32)],
    )(ex, s, batch, h5, d1)


def _k6_body(e_ref, m_ref, batch_ref, hw_ref, s_ref, ru_ref,
             accs_ref, accr_ref):
    i = pl.program_id(0)
    oh = _oh(batch_ref[...])
    m = m_ref[...]
    mc = jnp.where(m < -1e29, 0.0, m)
    mb = jnp.sum(oh * mc, axis=1, keepdims=True)
    ex = jnp.exp(e_ref[...] - mb)
    spart = jnp.sum(oh * ex, axis=0)[None]
    rupart = lax.dot_general(oh, ex * hw_ref[...],
                             (((0,), (0,)), ((), ())),
                             preferred_element_type=_f32, precision=lax.Precision.HIGHEST)

    @pl.when(i == 0)
    def _():
        accs_ref[...] = jnp.zeros((8, NG), _f32)
        accr_ref[...] = jnp.zeros((NG, H), _f32)

    accs_ref[0:1] += spart
    accr_ref[...] += rupart
    @pl.when(i == NBLK - 1)
    def _():
        s_ref[...] = accs_ref[0:1]
        ru_ref[...] = accr_ref[...]


def _k6(e, m, batch, hw):
    return pl.pallas_call(
        _k6_body,
        grid=(NBLK,),
        in_specs=[
            pl.BlockSpec((RB, 1), lambda i: (i, 0)),
            pl.BlockSpec((1, NG), lambda i: (0, 0)),
            pl.BlockSpec((RB, 1), lambda i: (i, 0)),
            pl.BlockSpec((RB, H), lambda i: (i, 0)),
        ],
        out_specs=[
            pl.BlockSpec((1, NG), lambda i: (0, 0)),
            pl.BlockSpec((NG, H), lambda i: (0, 0)),
        ],
        out_shape=[
            jax.ShapeDtypeStruct((1, NG), _f32),
            jax.ShapeDtypeStruct((NG, H), _f32),
        ],
        scratch_shapes=[pltpu.VMEM((8, NG), _f32), pltpu.VMEM((NG, H), _f32)],
    )(e, m, batch, hw)


def _k7_body(ru_ref, s_ref, hhp_ref, ccp_ref, wih_ref, whh_ref,
             bih_ref, bhh_ref, hh_ref, cc_ref):
    r = ru_ref[...] / (jnp.transpose(s_ref[...]) + 1e-16)
    hhp = hhp_ref[...]
    q = jnp.concatenate([hhp, r], axis=1)
    gates = (lax.dot_general(q, wih_ref[...], (((1,), (1,)), ((), ())),
                             preferred_element_type=_f32, precision=lax.Precision.HIGHEST) + bih_ref[...]
             + lax.dot_general(hhp, whh_ref[...], (((1,), (1,)), ((), ())),
                               preferred_element_type=_f32, precision=lax.Precision.HIGHEST) + bhh_ref[...])
    ii, ff, gg, oo = jnp.split(gates, 4, axis=1)
    cc = jax.nn.sigmoid(ff) * ccp_ref[...] + jax.nn.sigmoid(ii) * jnp.tanh(gg)
    hh = jax.nn.sigmoid(oo) * jnp.tanh(cc)
    hh_ref[...] = hh
    cc_ref[...] = cc


def _k7(ru, s, hh_prev, cc_prev, W_ih, W_hh, b_ih, b_hh):
    return pl.pallas_call(
        _k7_body,
        in_specs=[pl.BlockSpec(memory_space=pltpu.VMEM)] * 8,
        out_specs=[pl.BlockSpec(memory_space=pltpu.VMEM)] * 2,
        out_shape=[
            jax.ShapeDtypeStruct((NG, H), _f32),
            jax.ShapeDtypeStruct((NG, H), _f32),
        ],
    )(ru, s, hh_prev, cc_prev, W_ih, W_hh, b_ih, b_hh)


def _k8_body(hw_ref, batch_ref, hh_ref, e_ref, m_ref, acc_ref):
    i = pl.program_id(0)
    oh = _oh(batch_ref[...])
    hb = jnp.dot(oh, hh_ref[...], preferred_element_type=_f32, precision=lax.Precision.HIGHEST)
    e = jnp.sum(hw_ref[...] * hb, axis=1, keepdims=True)
    e_ref[...] = e
    masked = jnp.where(oh > 0, e, _NEG)
    mpart = jnp.max(masked, axis=0)[None]

    @pl.when(i == 0)
    def _():
        acc_ref[...] = jnp.full((8, NG), _NEG, _f32)

    acc_ref[0:1] = jnp.maximum(acc_ref[0:1], mpart)
    @pl.when(i == NBLK - 1)
    def _():
        m_ref[...] = acc_ref[0:1]


def _k8(hw, batch, hh):
    return pl.pallas_call(
        _k8_body,
        grid=(NBLK,),
        in_specs=[
            pl.BlockSpec((RB, H), lambda i: (i, 0)),
            pl.BlockSpec((RB, 1), lambda i: (i, 0)),
            pl.BlockSpec((NG, H), lambda i: (0, 0)),
        ],
        out_specs=[
            pl.BlockSpec((RB, 1), lambda i: (i, 0)),
            pl.BlockSpec((1, NG), lambda i: (0, 0)),
        ],
        out_shape=[
            jax.ShapeDtypeStruct((N, 1), _f32),
            jax.ShapeDtypeStruct((1, NG), _f32),
        ],
        scratch_shapes=[pltpu.VMEM((8, NG), _f32)],
    )(hw, batch, hh)


NPASS = 7


def _kt_body(score_ref, batch_ref, h_ref, m_ref, bih_ref, bhh_ref,
             wih_ref, whh_ref, wm_ref, bm_ref, wo1_ref, bo1_ref,
             wo2_ref, bo2_ref, out_ref, sacc_ref, macc_ref, stacc_ref,
             ruacc_ref):
    i = pl.program_id(0)
    j = lax.rem(i, NBLK)
    oh = _oh(batch_ref[...])
    m = m_ref[...]
    mc = jnp.where(m < -1e29, 0.0, m)
    ex = jnp.exp(score_ref[...] - jnp.sum(oh * mc, axis=1, keepdims=True))
    h5 = h_ref[...]
    bih = bih_ref[...]
    bhh = bhh_ref[...]
    wih = wih_ref[...]
    whh = whh_ref[...]

    gates0 = bih + bhh
    ii0, ff0, gg0, oo0 = jnp.split(gates0, 4, axis=1)
    cc1 = jax.nn.sigmoid(ii0) * jnp.tanh(gg0)
    hh1 = jax.nn.sigmoid(oo0) * jnp.tanh(cc1)

    def lstm(q, hhp, ccp):
        gates = (lax.dot_general(q, wih, (((1,), (1,)), ((), ())),
                                 preferred_element_type=_f32,
                                 precision=lax.Precision.HIGHEST) + bih
                 + lax.dot_general(hhp, whh, (((1,), (1,)), ((), ())),
                                   preferred_element_type=_f32,
                                   precision=lax.Precision.HIGHEST) + bhh)
        ii, ff, gg, oo = jnp.split(gates, 4, axis=1)
        cc = jax.nn.sigmoid(ff) * ccp + jax.nn.sigmoid(ii) * jnp.tanh(gg)
        hh = jax.nn.sigmoid(oo) * jnp.tanh(cc)
        return hh, cc

    def hh_chain(tmax):
        hh = jnp.broadcast_to(hh1, (NG, H))
        cc = jnp.broadcast_to(cc1, (NG, H))
        for t in range(1, tmax):
            ru = ruacc_ref[pl.ds((t - 1) * NG, NG), :]
            st = stacc_ref[t - 1:t]
            r = ru / (jnp.transpose(st) + 1e-16)
            q = jnp.concatenate([hh, r], axis=1)
            hh, cc = lstm(q, hh, cc)
        return hh, cc

    def nw_hw():
        sb = jnp.sum(oh * sacc_ref[0:1], axis=1, keepdims=True)
        nw = ex / (sb + 1e-16)
        return nw, h5 * nw

    def e_of(tmax, nw, hw):
        if tmax == 1:
            return nw * jnp.sum(h5 * hh1, axis=1, keepdims=True)
        hh, _ = hh_chain(tmax)
        return jnp.sum(hw * jnp.dot(oh, hh, preferred_element_type=_f32,
                                    precision=lax.Precision.HIGHEST),
                       axis=1, keepdims=True)

    def accum_max(t, e):
        @pl.when(j == 0)
        def _():
            macc_ref[t - 1:t] = jnp.full((1, NG), _NEG, _f32)
        macc_ref[t - 1:t] = jnp.maximum(
            macc_ref[t - 1:t], jnp.max(jnp.where(oh > 0, e, _NEG),
                                       axis=0)[None])

    def accum_sum_ru(t, e, hw):
        mt = macc_ref[t - 1:t]
        mtc = jnp.where(mt < -1e29, 0.0, mt)
        ext = jnp.exp(e - jnp.sum(oh * mtc, axis=1, keepdims=True))
        spart = jnp.sum(oh * ext, axis=0)[None]
        rupart = lax.dot_general(oh, ext * hw, (((0,), (0,)), ((), ())),
                                 preferred_element_type=_f32,
                                 precision=lax.Precision.HIGHEST)

        @pl.when(j == 0)
        def _():
            stacc_ref[t - 1:t] = jnp.zeros((1, NG), _f32)
            ruacc_ref[pl.ds((t - 1) * NG, NG), :] = jnp.zeros((NG, H), _f32)
        stacc_ref[t - 1:t] += spart
        ruacc_ref[pl.ds((t - 1) * NG, NG), :] += rupart

    @pl.when(i < NBLK)
    def _():
        @pl.when(j == 0)
        def _():
            sacc_ref[...] = jnp.zeros((1, NG), _f32)
        sacc_ref[...] += jnp.sum(oh * ex, axis=0)[None]

    @pl.when((i >= NBLK) & (i < 2 * NBLK))
    def _():
        nw, hw = nw_hw()
        accum_max(1, e_of(1, nw, hw))

    @pl.when((i >= 2 * NBLK) & (i < 3 * NBLK))
    def _():
        nw, hw = nw_hw()
        accum_sum_ru(1, e_of(1, nw, hw), hw)

    @pl.when((i >= 3 * NBLK) & (i < 4 * NBLK))
    def _():
        nw, hw = nw_hw()
        accum_max(2, e_of(2, nw, hw))

    @pl.when((i >= 4 * NBLK) & (i < 5 * NBLK))
    def _():
        nw, hw = nw_hw()
        accum_sum_ru(2, e_of(2, nw, hw), hw)

    @pl.when((i >= 5 * NBLK) & (i < 6 * NBLK))
    def _():
        nw, hw = nw_hw()
        accum_max(3, e_of(3, nw, hw))

    @pl.when(i >= 6 * NBLK)
    def _():
        nw, hw = nw_hw()
        accum_sum_ru(3, e_of(3, nw, hw), hw)

        @pl.when(i == NPASS * NBLK - 1)
        def _():
            hh3, _ = hh_chain(3)
            ru3 = ruacc_ref[pl.ds(2 * NG, NG), :]
            st3 = stacc_ref[2:3]
            r3 = ru3 / (jnp.transpose(st3) + 1e-16)
            q3 = jnp.concatenate([hh3, r3], axis=1)
            mm = jnp.maximum(
                jnp.dot(q3, wm_ref[...], preferred_element_type=_f32,
                        precision=lax.Precision.HIGHEST) + bm_ref[...], 0.0)
            mm = jnp.maximum(
                jnp.dot(mm, wo1_ref[...], preferred_element_type=_f32,
                        precision=lax.Precision.HIGHEST) + bo1_ref[...], 0.0)
            mm = jnp.dot(mm, wo2_ref[...], preferred_element_type=_f32,
                         precision=lax.Precision.HIGHEST) + bo2_ref[...]
            out_ref[...] = mm


def _kt(score, m, batch, h5, b_ih, b_hh, W_ih, W_hh,
        W_m, b_m, W_o1, b_o1, W_o2, b_o2):
    pj = lambda i: (lax.rem(i, NBLK), 0)
    c0 = lambda i: (0, 0)
    return pl.pallas_call(
        _kt_body,
        grid=(NPASS * NBLK,),
        in_specs=[
            pl.BlockSpec((RB, 1), pj),
            pl.BlockSpec((RB, 1), pj),
            pl.BlockSpec((RB, H), pj),
            pl.BlockSpec((1, NG), c0),
            pl.BlockSpec((1, 4 * H), c0),
            pl.BlockSpec((1, 4 * H), c0),
            pl.BlockSpec((4 * H, 2 * H), c0),
            pl.BlockSpec((4 * H, H), c0),
            pl.BlockSpec((2 * H, 2 * H), c0),
            pl.BlockSpec((1, 2 * H), c0),
            pl.BlockSpec((2 * H, 32), c0),
            pl.BlockSpec((1, 32), c0),
            pl.BlockSpec((32, 1), c0),
            pl.BlockSpec((1, 1), c0),
        ],
        out_specs=pl.BlockSpec((NG, 1), c0),
        out_shape=jax.ShapeDtypeStruct((NG, 1), _f32),
        scratch_shapes=[
            pltpu.VMEM((1, NG), _f32),
            pltpu.VMEM((8, NG), _f32),
            pltpu.VMEM((8, NG), _f32),
            pltpu.VMEM((3 * NG, H), _f32),
        ],
    )(score, batch, h5, m, b_ih, b_hh, W_ih, W_hh,
      W_m, b_m, W_o1, b_o1, W_o2, b_o2)


def _k13_body(ru_ref, s_ref, hhp_ref, wm_ref, bm_ref, wo1_ref, bo1_ref,
              wo2_ref, bo2_ref, out_ref):
    r = ru_ref[...] / (jnp.transpose(s_ref[...]) + 1e-16)
    hhp = hhp_ref[...]
    q = jnp.concatenate([hhp, r], axis=1)
    mm = jnp.maximum(jnp.dot(q, wm_ref[...],
                             preferred_element_type=_f32, precision=lax.Precision.HIGHEST) + bm_ref[...], 0.0)
    mm = jnp.maximum(jnp.dot(mm, wo1_ref[...],
                             preferred_element_type=_f32, precision=lax.Precision.HIGHEST) + bo1_ref[...], 0.0)
    mm = jnp.dot(mm, wo2_ref[...], preferred_element_type=_f32, precision=lax.Precision.HIGHEST) + bo2_ref[...]
    out_ref[...] = mm


def _k13(ru, s, hh_prev, W_m, b_m, W_o1, b_o1, W_o2, b_o2):
    return pl.pallas_call(
        _k13_body,
        in_specs=[pl.BlockSpec(memory_space=pltpu.VMEM)] * 9,
        out_specs=pl.BlockSpec(memory_space=pltpu.VMEM),
        out_shape=jax.ShapeDtypeStruct((NG, 1), _f32),
    )(ru, s, hh_prev, W_m, b_m, W_o1, b_o1, W_o2, b_o2)


E_PAD = 802816
EW = E_PAD // 32
CW = 1568
KE = 256
ET = E_PAD // 16
NA = 50176
NSL = NA // 16
ZB = 64
NP = 51200

_mesh = plsc.VectorSubcoreMesh(core_axis_name="c", subcore_axis_name="s")


def _zero16(buf, nrows, width):
    def z(i, _):
        for w in range(width // 16):
            buf[i, pl.ds(w * 16, 16)] = jnp.zeros((16,), _f32)
        return 0
    lax.fori_loop(0, nrows, z, 0)


def _deg_body(col_hbm, ew_hbm, out_hbm, degbuf, colb, ewb):
    cid = lax.axis_index("c")
    sid = lax.axis_index("s")
    wid = sid * 2 + cid

    def z(i, _):
        degbuf[pl.ds(i * 16, 16)] = jnp.zeros((16,), _f32)
        return 0
    lax.fori_loop(0, N // 16, z, 0)

    base = wid * EW
    lanes = lax.iota(jnp.int32, 16)
    for ch in range(16):
        pltpu.sync_copy(col_hbm.at[pl.ds(base + ch * CW, CW)], colb)
        pltpu.sync_copy(ew_hbm.at[pl.ds(base + ch * CW, CW)], ewb)

        def acc(i, _):
            cv = colb[pl.ds(i * 16, 16)]
            wv = ewb[pl.ds(i * 16, 16)]
            for k in range(16):
                plsc.addupdate_scatter(degbuf, [cv], wv, mask=lanes == k)
            return 0
        lax.fori_loop(0, CW // 16, acc, 0)
    pltpu.sync_copy(degbuf, out_hbm.at[wid, pl.ds(0, N)])


def _sc_deg(colp, ewp):
    f = pl.kernel(
        _deg_body,
        out_type=jax.ShapeDtypeStruct((32, NP), _f32),
        mesh=_mesh,
        compiler_params=pltpu.CompilerParams(use_tc_tiling_on_sc=False, needs_layout_passes=False),
        scratch_types=[
            pltpu.VMEM((N,), _f32),
            pltpu.VMEM((CW,), jnp.int32),
            pltpu.VMEM((CW,), _f32),
        ],
    )
    return f(colp, ewp)


def _norm_body(dis_hbm, row_hbm, col_hbm, ew_hbm, norm_hbm,
               disbuf, rowb, colb, ewb, nb):
    cid = lax.axis_index("c")
    sid = lax.axis_index("s")
    wid = sid * 2 + cid
    pltpu.sync_copy(dis_hbm, disbuf)
    base = wid * EW
    for ch in range(16):
        off = base + ch * CW
        pltpu.sync_copy(row_hbm.at[pl.ds(off, CW)], rowb)
        pltpu.sync_copy(col_hbm.at[pl.ds(off, CW)], colb)
        pltpu.sync_copy(ew_hbm.at[pl.ds(off, CW)], ewb)

        def grp(i, _):
            rv = rowb[pl.ds(i * 16, 16)]
            cv = colb[pl.ds(i * 16, 16)]
            dr = plsc.load_gather(disbuf, [rv])
            dc = plsc.load_gather(disbuf, [cv])
            nb[pl.ds(i * 16, 16)] = dr * ewb[pl.ds(i * 16, 16)] * dc
            return 0
        lax.fori_loop(0, CW // 16, grp, 0)
        pltpu.sync_copy(nb, norm_hbm.at[pl.ds(off, CW)])


def _sc_norm(dis_flat, rowp, colp, ewp):
    f = pl.kernel(
        _norm_body,
        out_type=jax.ShapeDtypeStruct((E_PAD,), _f32),
        mesh=_mesh,
        compiler_params=pltpu.CompilerParams(use_tc_tiling_on_sc=False, needs_layout_passes=False),
        scratch_types=[
            pltpu.VMEM((N,), _f32),
            pltpu.VMEM((CW,), jnp.int32),
            pltpu.VMEM((CW,), jnp.int32),
            pltpu.VMEM((CW,), _f32),
            pltpu.VMEM((CW,), _f32),
        ],
    )
    return f(dis_flat, rowp, colp, ewp)


NCH = ET // KE


def _scatter_body(xw_hbm, row_hbm, col_hbm, norm_hbm, out_hbm,
                  rowb0, colb0, normb0, rowb1, colb1, normb1,
                  gbuf0, gbuf1, sbuf, zbuf, acc,
                  isem0, isem1, gsem0, gsem1):
    cid = lax.axis_index("c")
    sid = lax.axis_index("s")
    rowb = (rowb0, rowb1)
    colb = (colb0, colb1)
    normb = (normb0, normb1)
    gbuf = (gbuf0, gbuf1)
    isem = (isem0, isem1)
    gsem = (gsem0, gsem1)
    base = sid * ET

    def start_idx(c, p):
        off = base + c * KE
        pltpu.async_copy(row_hbm.at[pl.ds(off, KE)], rowb[p], isem[p])
        pltpu.async_copy(col_hbm.at[pl.ds(off, KE)], colb[p], isem[p])
        pltpu.async_copy(norm_hbm.at[pl.ds(off, KE)], normb[p], isem[p])

    def drain_idx(p):
        pltpu.make_async_copy(row_hbm.at[pl.ds(0, KE)], rowb[p], isem[p]).wait()
        pltpu.make_async_copy(col_hbm.at[pl.ds(0, KE)], colb[p], isem[p]).wait()
        pltpu.make_async_copy(norm_hbm.at[pl.ds(0, KE)], normb[p], isem[p]).wait()

    def start_gather(p):
        pltpu.async_copy(xw_hbm.at[cid].at[rowb[p]], gbuf[p], gsem[p])

    def drain_gather(p):
        pltpu.make_async_copy(xw_hbm.at[cid].at[rowb[p]], gbuf[p],
                              gsem[p]).wait()

    def process(p):
        def scale(g, _):
            nvec = normb[p][pl.ds(g * 16, 16)]
            for k in range(16):
                i = g * 16 + k
                nv = jnp.full((16,), nvec[k], _f32)
                sbuf[i, pl.ds(0, 16)] = gbuf[p][i, pl.ds(0, 16)] * nv
                sbuf[i, pl.ds(16, 16)] = gbuf[p][i, pl.ds(16, 16)] * nv
            return 0
        lax.fori_loop(0, KE // 16, scale, 0)
        pltpu.sync_copy(sbuf, acc.at[colb[p]], add=True)

    _zero16(zbuf, ZB, 32)
    for z in range(NSL // ZB):
        pltpu.sync_copy(zbuf, acc.at[pl.ds(sid * NSL + z * ZB, ZB)])
    plsc.subcore_barrier()

    start_idx(0, 0)
    drain_idx(0)
    start_gather(0)
    start_idx(1, 1)

    def dchunk(jj, _):
        for p in (0, 1):
            c = jj * 2 + p
            q = 1 - p

            @pl.when(c + 1 < NCH)
            def _():
                drain_idx(q)
                start_gather(q)
            drain_gather(p)
            process(p)

            @pl.when(c + 2 < NCH)
            def _():
                start_idx(c + 2, p)
        return 0
    lax.fori_loop(0, NCH // 2, dchunk, 0)
    plsc.subcore_barrier()
    pltpu.sync_copy(acc.at[pl.ds(sid * NSL, NSL)],
                    out_hbm.at[cid].at[pl.ds(sid * NSL, NSL)])


def _sc_scatter(xw2, rowp, colp, normp):
    f = pl.kernel(
        _scatter_body,
        out_type=jax.ShapeDtypeStruct((2, NA, 32), _f32),
        mesh=_mesh,
        compiler_params=pltpu.CompilerParams(use_tc_tiling_on_sc=False, needs_layout_passes=False),
        scratch_types=[
            pltpu.VMEM((KE,), jnp.int32),
            pltpu.VMEM((KE,), jnp.int32),
            pltpu.VMEM((KE,), _f32),
            pltpu.VMEM((KE,), jnp.int32),
            pltpu.VMEM((KE,), jnp.int32),
            pltpu.VMEM((KE,), _f32),
            pltpu.VMEM((KE, 32), _f32),
            pltpu.VMEM((KE, 32), _f32),
            pltpu.VMEM((KE, 32), _f32),
            pltpu.VMEM((ZB, 32), _f32),
            pltpu.VMEM_SHARED((NA, 32), _f32),
            pltpu.SemaphoreType.DMA,
            pltpu.SemaphoreType.DMA,
            pltpu.SemaphoreType.DMA,
            pltpu.SemaphoreType.DMA,
        ],
    )
    return f(xw2, rowp, colp, normp)


def _k0_body(p_ref, deg_ref):
    deg_ref[...] = jnp.sum(p_ref[...], axis=0) + 2.0


def _k0(partials3):
    return pl.pallas_call(
        _k0_body,
        grid=(10,),
        in_specs=[pl.BlockSpec((32, 40, 128), lambda i: (0, i, 0))],
        out_specs=pl.BlockSpec((40, 128), lambda i: (i, 0)),
        out_shape=jax.ShapeDtypeStruct((400, 128), _f32),
    )(partials3)


def kernel(x, edge_index, edge_dist, global_info, batch,
           W_in, b_in, W_conv, b_conv, bn_g, bn_b, W_gin, b_gin,
           W_gls, b_gls, W_gout, b_gout, W_ih, W_hh, b_ih, b_hh,
           W_m, b_m, W_o1, b_o1, W_o2, b_o2):
    row, col = edge_index[0], edge_index[1]
    b_in2 = b_in.reshape(1, H)
    bn_g2 = bn_g.reshape(1, H)
    bn_b2 = bn_b.reshape(1, H)
    batch2 = batch.reshape(N, 1)
    b_ih2 = b_ih.reshape(1, 4 * H)
    b_hh2 = b_hh.reshape(1, 4 * H)

    padi = jnp.zeros((E_PAD - E,), jnp.int32)
    rowp = jnp.concatenate([row, padi])
    colp = jnp.concatenate([col, padi])
    ewp = jnp.concatenate([edge_dist, jnp.zeros((E_PAD - E,), _f32)])

    deg_part = _sc_deg(colp, ewp)
    deg = _k0(deg_part.reshape(32, 400, 128)).reshape(NP, 1)
    h0, xw, dis, sn = _k1(x, W_in, b_in2, W_conv[0], deg)
    norm = _sc_norm(dis.reshape(N), rowp, colp, ewp)

    prev = h0
    for i in range(5):
        scat = _sc_scatter(xw, rowp, colp, norm)
        out, sums = _k2(scat, xw, sn, b_conv[i].reshape(1, H))
        if i < 4:
            prev, xw = _k3(out, prev, sums, bn_g2, bn_b2, W_conv[i + 1])
        else:
            h5, score, m = _k3l(
                out, prev, sums, bn_g2, bn_b2, global_info, batch2,
                W_gin[:H], W_gin[H:], b_gin.reshape(1, H),
                W_gls[0], b_gls[0].reshape(1, H),
                W_gls[1], b_gls[1].reshape(1, H),
                W_gout, b_gout.reshape(1, 1))

    ex, s, d1 = _k4(score, batch2, m, h5, b_ih2, b_hh2)
    hw, e1, m1 = _k5(ex, s, batch2, h5, d1)
    s1, ru1 = _k6(e1, m1, batch2, hw)
    gates0 = (b_ih2 + b_hh2)
    ii0, ff0, gg0, oo0 = jnp.split(gates0, 4, axis=1)
    cc1 = jax.nn.sigmoid(ii0) * jnp.tanh(gg0)
    hh1 = jax.nn.sigmoid(oo0) * jnp.tanh(cc1)
    hh1f = jnp.broadcast_to(hh1, (NG, H))
    cc1f = jnp.broadcast_to(cc1, (NG, H))

    hh2, cc2 = _k7(ru1, s1, hh1f, cc1f, W_ih, W_hh, b_ih2, b_hh2)
    e2, m2 = _k8(hw, batch2, hh2)
    s2, ru2 = _k6(e2, m2, batch2, hw)
    hh3, _cc3 = _k7(ru2, s2, hh2, cc2, W_ih, W_hh, b_ih2, b_hh2)
    e3, m3 = _k8(hw, batch2, hh3)
    s3, ru3 = _k6(e3, m3, batch2, hw)
    out = _k13(ru3, s3, hh3, W_m, b_m.reshape(1, 2 * H),
               W_o1, b_o1.reshape(1, 32), W_o2, b_o2.reshape(1, 1))
    return out.reshape(1, NG)
